# Initial kernel scaffold; baseline (speedup 1.0000x reference)
import jax
import jax.numpy as jnp
from jax.experimental import pallas as pl


def kernel(uu_edges, ii_edges, ci_edges, user_table, item_table, Wu1, bu1, Wi1, bi1, Ws1, Wn1, bs1, Wu2, bu2, Wi2, bi2, Ws2, Wn2, bs2, Wu3, bu3, Wi3, bi3, Ws3, Wn3, bs3):
    raise NotImplementedError("write your pallas kernel here")


# speedup vs baseline: 10.8571x; 10.8571x over previous
import functools

import jax
import jax.numpy as jnp
from jax import lax
from jax.experimental import pallas as pl
from jax.experimental.pallas import tpu as pltpu
from jax.experimental.pallas import tpu_sc as plsc

N = 50000
D = 32
N_PAD = 50176
PAD_IDX = N
E_PHASE = 819200
ROWS_PHASE = 6400
ROWS_SUB = 400
IB = 40
NB = ROWS_SUB // IB
RING = 4
SLAB = N_PAD // 16
HIST_PAD = 65536
SLAB_H = HIST_PAD // 16

@functools.cache
def _mesh():
    return plsc.VectorSubcoreMesh(core_axis_name="c", subcore_axis_name="s",
                                  num_cores=2, num_subcores=16)


def _hist_body(hp1, hp2, hp3, zf, o1, o2, o3, hist, idx_ib, ones_v, sem):
    cid = lax.axis_index("c")
    sid = lax.axis_index("s")
    for k in range(8):
        ones_v[pl.ds(16 * k, 16)] = jnp.full((16,), 1.0, jnp.float32)

    def hphase(hp, out):
        pltpu.sync_copy(zf.at[pl.ds(sid * SLAB_H, SLAB_H)],
                        hist.at[pl.ds(sid * SLAB_H, SLAB_H)])
        plsc.subcore_barrier()
        base = cid * ROWS_PHASE + sid * ROWS_SUB
        for b in range(NB):
            pltpu.sync_copy(hp.at[pl.ds(base + b * IB, IB)], idx_ib)

            @pl.loop(0, IB)
            def _(j):
                pltpu.async_copy(ones_v, hist.at[idx_ib.at[j]], sem, add=True)

            @pl.loop(0, IB)
            def _(j):
                pltpu.make_async_copy(ones_v, hist.at[idx_ib.at[0]], sem).wait()
        plsc.subcore_barrier()
        pltpu.sync_copy(hist.at[pl.ds(sid * SLAB_H, SLAB_H)],
                        out.at[pl.ds(cid * HIST_PAD + sid * SLAB_H, SLAB_H)])

    hphase(hp1, o1)
    hphase(hp2, o2)
    hphase(hp3, o3)


_SC_PARAMS = pltpu.CompilerParams(use_tc_tiling_on_sc=False)


@functools.cache
def _hist_call():
    return pl.kernel(
        _hist_body,
        compiler_params=_SC_PARAMS,
        out_type=[jax.ShapeDtypeStruct((2 * HIST_PAD,), jnp.float32)] * 3,
        mesh=_mesh(),
        scratch_types=[
            pltpu.VMEM_SHARED((HIST_PAD,), jnp.float32),
            pltpu.VMEM((IB, 128), jnp.int32),
            pltpu.VMEM((128,), jnp.float32),
            pltpu.SemaphoreType.DMA,
        ],
    )


def _seg_body(h1, h2, p1s, p1d, p2s, p2d, z32, o1, o2,
              acc, sib, dib, rows, s0, s1, s2, s3):
    cid = lax.axis_index("c")
    sid = lax.axis_index("s")
    sems = (s0, s1, s2, s3)

    def sphase(h, ps, pd, out):
        pltpu.sync_copy(z32.at[pl.ds(sid * SLAB, SLAB)],
                        acc.at[pl.ds(sid * SLAB, SLAB)])
        plsc.subcore_barrier()
        base = cid * ROWS_PHASE + sid * ROWS_SUB
        for b in range(NB):
            row0 = base + b * IB
            pltpu.sync_copy(ps.at[pl.ds(row0, IB)], sib)
            pltpu.sync_copy(pd.at[pl.ds(row0, IB)], dib)
            for r in range(RING):
                pltpu.async_copy(h.at[sib.at[r]], rows.at[r], sems[r])

            @pl.loop(0, IB - RING, step=RING)
            def _(j):
                for r in range(RING):
                    pltpu.make_async_copy(
                        h.at[sib.at[0]], rows.at[r], sems[r]).wait()
                    pltpu.sync_copy(rows.at[r], acc.at[dib.at[j + r]],
                                    add=True)
                    pltpu.async_copy(h.at[sib.at[j + r + RING]], rows.at[r],
                                     sems[r])

            for r in range(RING):
                pltpu.make_async_copy(
                    h.at[sib.at[0]], rows.at[r], sems[r]).wait()
                pltpu.sync_copy(rows.at[r], acc.at[dib.at[IB - RING + r]],
                                add=True)
        plsc.subcore_barrier()
        pltpu.sync_copy(acc.at[pl.ds(sid * SLAB, SLAB)],
                        out.at[pl.ds(cid * N_PAD + sid * SLAB, SLAB)])

    sphase(h1, p1s, p1d, o1)
    sphase(h2, p2s, p2d, o2)


@functools.cache
def _seg_call():
    return pl.kernel(
        _seg_body,
        compiler_params=_SC_PARAMS,
        out_type=[jax.ShapeDtypeStruct((2 * N_PAD, D), jnp.float32)] * 2,
        mesh=_mesh(),
        scratch_types=[
            pltpu.VMEM_SHARED((N_PAD, D), jnp.float32),
            pltpu.VMEM((IB, 128), jnp.int32),
            pltpu.VMEM((IB, 128), jnp.int32),
            pltpu.VMEM((RING, 128, D), jnp.float32),
            pltpu.SemaphoreType.DMA,
            pltpu.SemaphoreType.DMA,
            pltpu.SemaphoreType.DMA,
            pltpu.SemaphoreType.DMA,
        ],
    )


_QROWS = 3136
_QGRID = N_PAD // _QROWS


def _norm_kernel_body(duo, dui, dio, dii, ca, cb, nuo, nui, nio, nii, ic):
    for d, o in ((duo, nuo), (dui, nui), (dio, nio), (dii, nii)):
        x = d[...]
        o[...] = jnp.where(x > 0, lax.rsqrt(x), 0.0)
    c = ca[...] + cb[...]
    ic[...] = 1.0 / jnp.maximum(c, 1.0)


def _norm_call(duo, dui, dio, dii, ca, cb):
    blk = pl.BlockSpec((N_PAD // 128, 128), lambda: (0, 0))
    return pl.pallas_call(
        _norm_kernel_body,
        out_shape=[jax.ShapeDtypeStruct((N_PAD // 128, 128), jnp.float32)] * 5,
        in_specs=[blk] * 6,
        out_specs=[blk] * 5,
    )(duo, dui, dio, dii, ca, cb)


def _pre_body(u, i, nuo, nio, hus, his):
    hus[...] = u[...] * nuo[...]
    his[...] = i[...] * nio[...]


def _pre_call(u, i, nuo, nio):
    fb = pl.BlockSpec((_QROWS, D), lambda r: (r, 0))
    nb = pl.BlockSpec((_QROWS, 1), lambda r: (r, 0))
    return pl.pallas_call(
        _pre_body,
        grid=(_QGRID,),
        out_shape=[jax.ShapeDtypeStruct((N_PAD, D), jnp.float32)] * 2,
        in_specs=[fb, fb, nb, nb],
        out_specs=[fb, fb],
    )(u, i, nuo, nio)


def _q_body(final, su, si, ca, cb, hi_in, niu, nii, ic, nuo, nio,
            Wu, bu, Wi, bi, Ws, Wn, bs, *outs):
    f32 = jnp.float32
    ou = jnp.dot(su[...], Wu[...], preferred_element_type=f32) * niu[...] \
        + bu[...]
    mean = (ca[...] + cb[...]) * ic[...]
    oi = jnp.dot(si[...], Wi[...], preferred_element_type=f32) * nii[...] \
        + bi[...] \
        + jnp.dot(hi_in[...], Ws[...], preferred_element_type=f32) \
        + jnp.dot(mean, Wn[...], preferred_element_type=f32) + bs[...]
    if final:
        outs[0][...] = ou
        outs[1][...] = oi
    else:
        hu2 = jnp.maximum(ou, 0.0)
        hi2 = jnp.maximum(oi, 0.0)
        outs[0][...] = hu2
        outs[1][...] = hi2
        outs[2][...] = hu2 * nuo[...]
        outs[3][...] = hi2 * nio[...]


def _q_call(final, su, si, ca, cb, hi_in, niu, nii, ic, nuo, nio,
            Wu, bu, Wi, bi, Ws, Wn, bs):
    fb = pl.BlockSpec((_QROWS, D), lambda r: (r, 0))
    nb = pl.BlockSpec((_QROWS, 1), lambda r: (r, 0))
    wb = pl.BlockSpec((D, D), lambda r: (0, 0))
    bb = pl.BlockSpec((1, D), lambda r: (0, 0))
    n_out = 2 if final else 4
    return pl.pallas_call(
        functools.partial(_q_body, final),
        grid=(_QGRID,),
        out_shape=[jax.ShapeDtypeStruct((N_PAD, D), jnp.float32)] * n_out,
        in_specs=[fb, fb, fb, fb, fb, nb, nb, nb, nb, nb,
                  wb, bb, wb, bb, wb, wb, bb],
        out_specs=[fb] * n_out,
    )(su, si, ca, cb, hi_in, niu, nii, ic, nuo, nio,
      Wu, bu, Wi, bi, Ws, Wn, bs)


def _pad_idx(a, e_pad):
    pad = e_pad - a.shape[0]
    return jnp.concatenate([a, jnp.full((pad,), PAD_IDX, jnp.int32)])


def _pad_rows(t):
    return jnp.concatenate([t, jnp.zeros((N_PAD - N, D), t.dtype)])


def kernel(uu_edges, ii_edges, ci_edges, user_table, item_table,
           Wu1, bu1, Wi1, bi1, Ws1, Wn1, bs1,
           Wu2, bu2, Wi2, bi2, Ws2, Wn2, bs2,
           Wu3, bu3, Wi3, bi3, Ws3, Wn3, bs3):
    f32 = jnp.float32
    up = _pad_rows(user_table)
    ip = _pad_rows(item_table)
    z32 = jnp.zeros((N_PAD, D), f32)
    zf = jnp.zeros((HIST_PAD,), f32)

    uu_s = _pad_idx(uu_edges[0], E_PHASE)
    uu_d = _pad_idx(uu_edges[1], E_PHASE)
    ii_s = _pad_idx(ii_edges[0], E_PHASE)
    ii_d = _pad_idx(ii_edges[1], E_PHASE)
    ci_s = _pad_idx(ci_edges[0], 2 * E_PHASE)
    ci_d = _pad_idx(ci_edges[1], 2 * E_PHASE)
    ca_s, cb_s = ci_s[:E_PHASE], ci_s[E_PHASE:]
    ca_d, cb_d = ci_d[:E_PHASE], ci_d[E_PHASE:]

    def r2(a):
        return a.reshape(ROWS_PHASE, 128)

    def stack(a, b):
        return jnp.concatenate([r2(a), r2(b)])

    p1s = stack(uu_s, ca_s + N_PAD)
    p1d = stack(uu_d, ca_d)
    p2s = stack(ii_s, cb_s + N_PAD)
    p2d = stack(ii_d, cb_d)
    hp1 = stack(uu_s, ii_d)
    hp2 = stack(uu_d, ca_d)
    hp3 = stack(ii_s, cb_d)

    h1o, h2o, h3o = _hist_call()(hp1, hp2, hp3, zf)
    dq = N_PAD // 128

    def rq(a):
        return a.reshape(dq, 128)

    nuo, nui, nio, nii, ic = _norm_call(
        rq(h1o[:N_PAD]), rq(h2o[:N_PAD]), rq(h3o[:N_PAD]),
        rq(h1o[HIST_PAD:HIST_PAD + N_PAD]),
        rq(h2o[HIST_PAD:HIST_PAD + N_PAD]),
        rq(h3o[HIST_PAD:HIST_PAD + N_PAD]))

    def rcol(a):
        return a.reshape(N_PAD, 1)

    nuo, nui, nio, nii, ic = map(rcol, (nuo, nui, nio, nii, ic))

    hus, his = _pre_call(up, ip, nuo, nio)
    hu, hi = up, ip
    params = [(Wu1, bu1, Wi1, bi1, Ws1, Wn1, bs1),
              (Wu2, bu2, Wi2, bi2, Ws2, Wn2, bs2),
              (Wu3, bu3, Wi3, bi3, Ws3, Wn3, bs3)]
    for l, (Wu, bu, Wi, bi, Ws, Wn, bs) in enumerate(params):
        h1 = jnp.concatenate([hus, hu])
        h2 = jnp.concatenate([his, hu])
        o1, o2 = _seg_call()(h1, h2, p1s, p1d, p2s, p2d, z32)
        su, sca = o1[:N_PAD], o1[N_PAD:]
        si, scb = o2[:N_PAD], o2[N_PAD:]
        final = l == 2
        res = _q_call(final, su, si, sca, scb, hi, nui, nii, ic, nuo, nio,
                      Wu, bu.reshape(1, D), Wi, bi.reshape(1, D),
                      Ws, Wn, bs.reshape(1, D))
        if final:
            return res[0][:N], res[1][:N]
        hu, hi, hus, his = res


# speedup vs baseline: 10.8809x; 1.0022x over previous
import functools

import jax
import jax.numpy as jnp
from jax import lax
from jax.experimental import pallas as pl
from jax.experimental.pallas import tpu as pltpu
from jax.experimental.pallas import tpu_sc as plsc

N = 50000
D = 32
N_PAD = 50176
PAD_IDX = N
E_PHASE = 819200
ROWS_PHASE = 6400
ROWS_SUB = 400
IB = 40
NB = ROWS_SUB // IB
RING = 4
SLAB = N_PAD // 16
HIST_PAD = 65536
SLAB_H = HIST_PAD // 16

@functools.cache
def _mesh():
    return plsc.VectorSubcoreMesh(core_axis_name="c", subcore_axis_name="s",
                                  num_cores=2, num_subcores=16)


def _hist_body(hp1, hp2, hp3, zf, o1, o2, o3, hist, idx_ib, ones_v, sem):
    cid = lax.axis_index("c")
    sid = lax.axis_index("s")
    for k in range(8):
        ones_v[pl.ds(16 * k, 16)] = jnp.full((16,), 1.0, jnp.float32)

    def hphase(hp, out):
        pltpu.sync_copy(zf.at[pl.ds(sid * SLAB_H, SLAB_H)],
                        hist.at[pl.ds(sid * SLAB_H, SLAB_H)])
        plsc.subcore_barrier()
        base = cid * ROWS_PHASE + sid * ROWS_SUB

        @pl.loop(0, NB)
        def _(b):
            pltpu.sync_copy(hp.at[pl.ds(base + b * IB, IB)], idx_ib)

            @pl.loop(0, IB)
            def _(j):
                pltpu.async_copy(ones_v, hist.at[idx_ib.at[j]], sem, add=True)

            @pl.loop(0, IB)
            def _(j):
                pltpu.make_async_copy(ones_v, hist.at[idx_ib.at[0]], sem).wait()
        plsc.subcore_barrier()
        pltpu.sync_copy(hist.at[pl.ds(sid * SLAB_H, SLAB_H)],
                        out.at[pl.ds(cid * HIST_PAD + sid * SLAB_H, SLAB_H)])

    hphase(hp1, o1)
    hphase(hp2, o2)
    hphase(hp3, o3)


_SC_PARAMS = pltpu.CompilerParams(use_tc_tiling_on_sc=False)


@functools.cache
def _hist_call():
    return pl.kernel(
        _hist_body,
        compiler_params=_SC_PARAMS,
        out_type=[jax.ShapeDtypeStruct((2 * HIST_PAD,), jnp.float32)] * 3,
        mesh=_mesh(),
        scratch_types=[
            pltpu.VMEM_SHARED((HIST_PAD,), jnp.float32),
            pltpu.VMEM((IB, 128), jnp.int32),
            pltpu.VMEM((128,), jnp.float32),
            pltpu.SemaphoreType.DMA,
        ],
    )


def _seg_body(h1, h2, p1s, p1d, p2s, p2d, z32, o1, o2,
              acc, sib, dib, rows, s0, s1, s2, s3):
    cid = lax.axis_index("c")
    sid = lax.axis_index("s")
    sems = (s0, s1, s2, s3)

    def sphase(h, ps, pd, out):
        pltpu.sync_copy(z32.at[pl.ds(sid * SLAB, SLAB)],
                        acc.at[pl.ds(sid * SLAB, SLAB)])
        plsc.subcore_barrier()
        base = cid * ROWS_PHASE + sid * ROWS_SUB

        @pl.loop(0, NB)
        def _(b):
            row0 = base + b * IB
            pltpu.sync_copy(ps.at[pl.ds(row0, IB)], sib)
            pltpu.sync_copy(pd.at[pl.ds(row0, IB)], dib)
            for r in range(RING):
                pltpu.async_copy(h.at[sib.at[r]], rows.at[r], sems[r])

            @pl.loop(0, IB - RING, step=RING)
            def _(j):
                for r in range(RING):
                    pltpu.make_async_copy(
                        h.at[sib.at[0]], rows.at[r], sems[r]).wait()
                    pltpu.sync_copy(rows.at[r], acc.at[dib.at[j + r]],
                                    add=True)
                    pltpu.async_copy(h.at[sib.at[j + r + RING]], rows.at[r],
                                     sems[r])

            for r in range(RING):
                pltpu.make_async_copy(
                    h.at[sib.at[0]], rows.at[r], sems[r]).wait()
                pltpu.sync_copy(rows.at[r], acc.at[dib.at[IB - RING + r]],
                                add=True)
        plsc.subcore_barrier()
        pltpu.sync_copy(acc.at[pl.ds(sid * SLAB, SLAB)],
                        out.at[pl.ds(cid * N_PAD + sid * SLAB, SLAB)])

    sphase(h1, p1s, p1d, o1)
    sphase(h2, p2s, p2d, o2)


@functools.cache
def _seg_call():
    return pl.kernel(
        _seg_body,
        compiler_params=_SC_PARAMS,
        out_type=[jax.ShapeDtypeStruct((2 * N_PAD, D), jnp.float32)] * 2,
        mesh=_mesh(),
        scratch_types=[
            pltpu.VMEM_SHARED((N_PAD, D), jnp.float32),
            pltpu.VMEM((IB, 128), jnp.int32),
            pltpu.VMEM((IB, 128), jnp.int32),
            pltpu.VMEM((RING, 128, D), jnp.float32),
            pltpu.SemaphoreType.DMA,
            pltpu.SemaphoreType.DMA,
            pltpu.SemaphoreType.DMA,
            pltpu.SemaphoreType.DMA,
        ],
    )


_QROWS = 3136
_QGRID = N_PAD // _QROWS


def _norm_kernel_body(duo, dui, dio, dii, ca, cb, nuo, nui, nio, nii, ic):
    for d, o in ((duo, nuo), (dui, nui), (dio, nio), (dii, nii)):
        x = d[...]
        o[...] = jnp.where(x > 0, lax.rsqrt(x), 0.0)
    c = ca[...] + cb[...]
    ic[...] = 1.0 / jnp.maximum(c, 1.0)


def _norm_call(duo, dui, dio, dii, ca, cb):
    blk = pl.BlockSpec((N_PAD // 128, 128), lambda: (0, 0))
    return pl.pallas_call(
        _norm_kernel_body,
        out_shape=[jax.ShapeDtypeStruct((N_PAD // 128, 128), jnp.float32)] * 5,
        in_specs=[blk] * 6,
        out_specs=[blk] * 5,
    )(duo, dui, dio, dii, ca, cb)


def _pre_body(u, i, nuo, nio, hus, his):
    hus[...] = u[...] * nuo[...]
    his[...] = i[...] * nio[...]


def _pre_call(u, i, nuo, nio):
    fb = pl.BlockSpec((_QROWS, D), lambda r: (r, 0))
    nb = pl.BlockSpec((_QROWS, 1), lambda r: (r, 0))
    return pl.pallas_call(
        _pre_body,
        grid=(_QGRID,),
        out_shape=[jax.ShapeDtypeStruct((N_PAD, D), jnp.float32)] * 2,
        in_specs=[fb, fb, nb, nb],
        out_specs=[fb, fb],
    )(u, i, nuo, nio)


def _q_body(final, su, si, ca, cb, hi_in, niu, nii, ic, nuo, nio,
            Wu, bu, Wi, bi, Ws, Wn, bs, *outs):
    f32 = jnp.float32
    ou = jnp.dot(su[...], Wu[...], preferred_element_type=f32) * niu[...] \
        + bu[...]
    mean = (ca[...] + cb[...]) * ic[...]
    oi = jnp.dot(si[...], Wi[...], preferred_element_type=f32) * nii[...] \
        + bi[...] \
        + jnp.dot(hi_in[...], Ws[...], preferred_element_type=f32) \
        + jnp.dot(mean, Wn[...], preferred_element_type=f32) + bs[...]
    if final:
        outs[0][...] = ou
        outs[1][...] = oi
    else:
        hu2 = jnp.maximum(ou, 0.0)
        hi2 = jnp.maximum(oi, 0.0)
        outs[0][...] = hu2
        outs[1][...] = hi2
        outs[2][...] = hu2 * nuo[...]
        outs[3][...] = hi2 * nio[...]


def _q_call(final, su, si, ca, cb, hi_in, niu, nii, ic, nuo, nio,
            Wu, bu, Wi, bi, Ws, Wn, bs):
    fb = pl.BlockSpec((_QROWS, D), lambda r: (r, 0))
    nb = pl.BlockSpec((_QROWS, 1), lambda r: (r, 0))
    wb = pl.BlockSpec((D, D), lambda r: (0, 0))
    bb = pl.BlockSpec((1, D), lambda r: (0, 0))
    n_out = 2 if final else 4
    return pl.pallas_call(
        functools.partial(_q_body, final),
        grid=(_QGRID,),
        out_shape=[jax.ShapeDtypeStruct((N_PAD, D), jnp.float32)] * n_out,
        in_specs=[fb, fb, fb, fb, fb, nb, nb, nb, nb, nb,
                  wb, bb, wb, bb, wb, wb, bb],
        out_specs=[fb] * n_out,
    )(su, si, ca, cb, hi_in, niu, nii, ic, nuo, nio,
      Wu, bu, Wi, bi, Ws, Wn, bs)


def _pad_idx(a, e_pad):
    pad = e_pad - a.shape[0]
    return jnp.concatenate([a, jnp.full((pad,), PAD_IDX, jnp.int32)])


def _pad_rows(t):
    return jnp.concatenate([t, jnp.zeros((N_PAD - N, D), t.dtype)])


def kernel(uu_edges, ii_edges, ci_edges, user_table, item_table,
           Wu1, bu1, Wi1, bi1, Ws1, Wn1, bs1,
           Wu2, bu2, Wi2, bi2, Ws2, Wn2, bs2,
           Wu3, bu3, Wi3, bi3, Ws3, Wn3, bs3):
    f32 = jnp.float32
    up = _pad_rows(user_table)
    ip = _pad_rows(item_table)
    z32 = jnp.zeros((N_PAD, D), f32)
    zf = jnp.zeros((HIST_PAD,), f32)

    uu_s = _pad_idx(uu_edges[0], E_PHASE)
    uu_d = _pad_idx(uu_edges[1], E_PHASE)
    ii_s = _pad_idx(ii_edges[0], E_PHASE)
    ii_d = _pad_idx(ii_edges[1], E_PHASE)
    ci_s = _pad_idx(ci_edges[0], 2 * E_PHASE)
    ci_d = _pad_idx(ci_edges[1], 2 * E_PHASE)
    ca_s, cb_s = ci_s[:E_PHASE], ci_s[E_PHASE:]
    ca_d, cb_d = ci_d[:E_PHASE], ci_d[E_PHASE:]

    def r2(a):
        return a.reshape(ROWS_PHASE, 128)

    def stack(a, b):
        return jnp.concatenate([r2(a), r2(b)])

    p1s = stack(uu_s, ca_s + N_PAD)
    p1d = stack(uu_d, ca_d)
    p2s = stack(ii_s, cb_s + N_PAD)
    p2d = stack(ii_d, cb_d)
    hp1 = stack(uu_s, ii_d)
    hp2 = stack(uu_d, ca_d)
    hp3 = stack(ii_s, cb_d)

    h1o, h2o, h3o = _hist_call()(hp1, hp2, hp3, zf)
    dq = N_PAD // 128

    def rq(a):
        return a.reshape(dq, 128)

    nuo, nui, nio, nii, ic = _norm_call(
        rq(h1o[:N_PAD]), rq(h2o[:N_PAD]), rq(h3o[:N_PAD]),
        rq(h1o[HIST_PAD:HIST_PAD + N_PAD]),
        rq(h2o[HIST_PAD:HIST_PAD + N_PAD]),
        rq(h3o[HIST_PAD:HIST_PAD + N_PAD]))

    def rcol(a):
        return a.reshape(N_PAD, 1)

    nuo, nui, nio, nii, ic = map(rcol, (nuo, nui, nio, nii, ic))

    hus, his = _pre_call(up, ip, nuo, nio)
    hu, hi = up, ip
    params = [(Wu1, bu1, Wi1, bi1, Ws1, Wn1, bs1),
              (Wu2, bu2, Wi2, bi2, Ws2, Wn2, bs2),
              (Wu3, bu3, Wi3, bi3, Ws3, Wn3, bs3)]
    for l, (Wu, bu, Wi, bi, Ws, Wn, bs) in enumerate(params):
        h1 = jnp.concatenate([hus, hu])
        h2 = jnp.concatenate([his, hu])
        o1, o2 = _seg_call()(h1, h2, p1s, p1d, p2s, p2d, z32)
        su, sca = o1[:N_PAD], o1[N_PAD:]
        si, scb = o2[:N_PAD], o2[N_PAD:]
        final = l == 2
        res = _q_call(final, su, si, sca, scb, hi, nui, nii, ic, nuo, nio,
                      Wu, bu.reshape(1, D), Wi, bi.reshape(1, D),
                      Ws, Wn, bs.reshape(1, D))
        if final:
            return res[0][:N], res[1][:N]
        hu, hi, hus, his = res


# speedup vs baseline: 11.2999x; 1.0385x over previous
import functools

import jax
import jax.numpy as jnp
from jax import lax
from jax.experimental import pallas as pl
from jax.experimental.pallas import tpu as pltpu
from jax.experimental.pallas import tpu_sc as plsc

N = 50000
D = 32
N_PAD = 50048
PAD_IDX = N
E_PHASE = 819200
ROWS_PHASE = 6400
ROWS_SUB = 400
IB = 40
NB = ROWS_SUB // IB
RING = 5
SLAB = N_PAD // 16
HIST_PAD = 65536
SLAB_H = HIST_PAD // 16

@functools.cache
def _mesh():
    return plsc.VectorSubcoreMesh(core_axis_name="c", subcore_axis_name="s",
                                  num_cores=2, num_subcores=16)


def _hist_body(hp1, hp2, hp3, zf, o1, o2, o3, hist, idx_ib, ones_v, sem):
    cid = lax.axis_index("c")
    sid = lax.axis_index("s")
    for k in range(8):
        ones_v[pl.ds(16 * k, 16)] = jnp.full((16,), 1.0, jnp.float32)

    def hphase(hp, out):
        pltpu.sync_copy(zf.at[pl.ds(sid * SLAB_H, SLAB_H)],
                        hist.at[pl.ds(sid * SLAB_H, SLAB_H)])
        plsc.subcore_barrier()
        base = cid * ROWS_PHASE + sid * ROWS_SUB

        @pl.loop(0, NB)
        def _(b):
            pltpu.sync_copy(hp.at[pl.ds(base + b * IB, IB)], idx_ib)

            @pl.loop(0, IB)
            def _(j):
                pltpu.async_copy(ones_v, hist.at[idx_ib.at[j]], sem, add=True)

            @pl.loop(0, IB)
            def _(j):
                pltpu.make_async_copy(ones_v, hist.at[idx_ib.at[0]], sem).wait()
        plsc.subcore_barrier()
        pltpu.sync_copy(hist.at[pl.ds(sid * SLAB_H, SLAB_H)],
                        out.at[pl.ds(cid * HIST_PAD + sid * SLAB_H, SLAB_H)])

    hphase(hp1, o1)
    hphase(hp2, o2)
    hphase(hp3, o3)


_SC_PARAMS = pltpu.CompilerParams(use_tc_tiling_on_sc=False)


@functools.cache
def _hist_call():
    return pl.kernel(
        _hist_body,
        compiler_params=_SC_PARAMS,
        out_type=[jax.ShapeDtypeStruct((2 * HIST_PAD,), jnp.float32)] * 3,
        mesh=_mesh(),
        scratch_types=[
            pltpu.VMEM_SHARED((HIST_PAD,), jnp.float32),
            pltpu.VMEM((IB, 128), jnp.int32),
            pltpu.VMEM((128,), jnp.float32),
            pltpu.SemaphoreType.DMA,
        ],
    )


def _seg_body(h1, h2, p1s, p1d, p2s, p2d, z32, o1, o2,
              acc, sib, dib, rows,
              g0, g1, g2, g3, g4, t0, t1, t2, t3, t4):
    cid = lax.axis_index("c")
    sid = lax.axis_index("s")
    gs = (g0, g1, g2, g3, g4)
    ss = (t0, t1, t2, t3, t4)

    def sphase(h, ps, pd, out):
        pltpu.sync_copy(z32.at[pl.ds(sid * SLAB, SLAB)],
                        acc.at[pl.ds(sid * SLAB, SLAB)])
        plsc.subcore_barrier()
        base = cid * ROWS_PHASE + sid * ROWS_SUB

        @pl.loop(0, NB)
        def _(b):
            row0 = base + b * IB
            pltpu.sync_copy(ps.at[pl.ds(row0, IB)], sib)
            pltpu.sync_copy(pd.at[pl.ds(row0, IB)], dib)
            for c in range(3):
                pltpu.async_copy(h.at[sib.at[c]], rows.at[c], gs[c])
            for jj in (0, 1):
                pltpu.async_copy(h.at[sib.at[jj + 3]], rows.at[jj + 3],
                                 gs[jj + 3])
                pltpu.make_async_copy(h.at[sib.at[0]], rows.at[jj],
                                      gs[jj]).wait()
                pltpu.async_copy(rows.at[jj], acc.at[dib.at[jj]], ss[jj],
                                 add=True)

            @pl.loop(2, IB - 3, step=5)
            def _(j):
                for off in range(5):
                    s = (2 + off) % 5
                    pltpu.make_async_copy(rows.at[off], acc.at[dib.at[0]],
                                          ss[off]).wait()
                    pltpu.async_copy(h.at[sib.at[j + off + 3]], rows.at[off],
                                     gs[off])
                    pltpu.make_async_copy(h.at[sib.at[0]], rows.at[s],
                                          gs[s]).wait()
                    pltpu.async_copy(rows.at[s], acc.at[dib.at[j + off]],
                                     ss[s], add=True)

            for jj in (IB - 3, IB - 2, IB - 1):
                s = jj % 5
                pltpu.make_async_copy(h.at[sib.at[0]], rows.at[s],
                                      gs[s]).wait()
                pltpu.async_copy(rows.at[s], acc.at[dib.at[jj]], ss[s],
                                 add=True)
            for r in range(5):
                pltpu.make_async_copy(rows.at[r], acc.at[dib.at[0]],
                                      ss[r]).wait()
        plsc.subcore_barrier()
        pltpu.sync_copy(acc.at[pl.ds(sid * SLAB, SLAB)],
                        out.at[pl.ds(cid * N_PAD + sid * SLAB, SLAB)])

    sphase(h1, p1s, p1d, o1)
    sphase(h2, p2s, p2d, o2)


@functools.cache
def _seg_call():
    return pl.kernel(
        _seg_body,
        compiler_params=_SC_PARAMS,
        out_type=[jax.ShapeDtypeStruct((2 * N_PAD, D), jnp.float32)] * 2,
        mesh=_mesh(),
        scratch_types=[
            pltpu.VMEM_SHARED((N_PAD, D), jnp.float32),
            pltpu.VMEM((IB, 128), jnp.int32),
            pltpu.VMEM((IB, 128), jnp.int32),
            pltpu.VMEM((RING, 128, D), jnp.float32),
        ] + [pltpu.SemaphoreType.DMA] * 10,
    )


_QROWS = N_PAD // 16
_QGRID = N_PAD // _QROWS


def _norm_kernel_body(duo, dui, dio, dii, ca, cb, nuo, nui, nio, nii, ic):
    for d, o in ((duo, nuo), (dui, nui), (dio, nio), (dii, nii)):
        x = d[...]
        o[...] = jnp.where(x > 0, lax.rsqrt(x), 0.0)
    c = ca[...] + cb[...]
    ic[...] = 1.0 / jnp.maximum(c, 1.0)


def _norm_call(duo, dui, dio, dii, ca, cb):
    blk = pl.BlockSpec((N_PAD // 128, 128), lambda: (0, 0))
    return pl.pallas_call(
        _norm_kernel_body,
        out_shape=[jax.ShapeDtypeStruct((N_PAD // 128, 128), jnp.float32)] * 5,
        in_specs=[blk] * 6,
        out_specs=[blk] * 5,
    )(duo, dui, dio, dii, ca, cb)


def _pre_body(u, i, nuo, nio, hus, his):
    hus[...] = u[...] * nuo[...]
    his[...] = i[...] * nio[...]


def _pre_call(u, i, nuo, nio):
    fb = pl.BlockSpec((_QROWS, D), lambda r: (r, 0))
    nb = pl.BlockSpec((_QROWS, 1), lambda r: (r, 0))
    return pl.pallas_call(
        _pre_body,
        grid=(_QGRID,),
        out_shape=[jax.ShapeDtypeStruct((N_PAD, D), jnp.float32)] * 2,
        in_specs=[fb, fb, nb, nb],
        out_specs=[fb, fb],
    )(u, i, nuo, nio)


def _q_body(final, su, si, ca, cb, hi_in, niu, nii, ic, nuo, nio,
            Wu, bu, Wi, bi, Ws, Wn, bs, *outs):
    f32 = jnp.float32
    ou = jnp.dot(su[...], Wu[...], preferred_element_type=f32) * niu[...] \
        + bu[...]
    mean = (ca[...] + cb[...]) * ic[...]
    oi = jnp.dot(si[...], Wi[...], preferred_element_type=f32) * nii[...] \
        + bi[...] \
        + jnp.dot(hi_in[...], Ws[...], preferred_element_type=f32) \
        + jnp.dot(mean, Wn[...], preferred_element_type=f32) + bs[...]
    if final:
        outs[0][...] = ou
        outs[1][...] = oi
    else:
        hu2 = jnp.maximum(ou, 0.0)
        hi2 = jnp.maximum(oi, 0.0)
        outs[0][...] = hu2
        outs[1][...] = hi2
        outs[2][...] = hu2 * nuo[...]
        outs[3][...] = hi2 * nio[...]


def _q_call(final, su, si, ca, cb, hi_in, niu, nii, ic, nuo, nio,
            Wu, bu, Wi, bi, Ws, Wn, bs):
    fb = pl.BlockSpec((_QROWS, D), lambda r: (r, 0))
    nb = pl.BlockSpec((_QROWS, 1), lambda r: (r, 0))
    wb = pl.BlockSpec((D, D), lambda r: (0, 0))
    bb = pl.BlockSpec((1, D), lambda r: (0, 0))
    n_out = 2 if final else 4
    return pl.pallas_call(
        functools.partial(_q_body, final),
        grid=(_QGRID,),
        out_shape=[jax.ShapeDtypeStruct((N_PAD, D), jnp.float32)] * n_out,
        in_specs=[fb, fb, fb, fb, fb, nb, nb, nb, nb, nb,
                  wb, bb, wb, bb, wb, wb, bb],
        out_specs=[fb] * n_out,
    )(su, si, ca, cb, hi_in, niu, nii, ic, nuo, nio,
      Wu, bu, Wi, bi, Ws, Wn, bs)


def _pad_idx(a, e_pad):
    pad = e_pad - a.shape[0]
    return jnp.concatenate([a, jnp.full((pad,), PAD_IDX, jnp.int32)])


def _pad_rows(t):
    return jnp.concatenate([t, jnp.zeros((N_PAD - N, D), t.dtype)])


def kernel(uu_edges, ii_edges, ci_edges, user_table, item_table,
           Wu1, bu1, Wi1, bi1, Ws1, Wn1, bs1,
           Wu2, bu2, Wi2, bi2, Ws2, Wn2, bs2,
           Wu3, bu3, Wi3, bi3, Ws3, Wn3, bs3):
    f32 = jnp.float32
    up = _pad_rows(user_table)
    ip = _pad_rows(item_table)
    z32 = jnp.zeros((N_PAD, D), f32)
    zf = jnp.zeros((HIST_PAD,), f32)

    uu_s = _pad_idx(uu_edges[0], E_PHASE)
    uu_d = _pad_idx(uu_edges[1], E_PHASE)
    ii_s = _pad_idx(ii_edges[0], E_PHASE)
    ii_d = _pad_idx(ii_edges[1], E_PHASE)
    ci_s = _pad_idx(ci_edges[0], 2 * E_PHASE)
    ci_d = _pad_idx(ci_edges[1], 2 * E_PHASE)
    ca_s, cb_s = ci_s[:E_PHASE], ci_s[E_PHASE:]
    ca_d, cb_d = ci_d[:E_PHASE], ci_d[E_PHASE:]

    def r2(a):
        return a.reshape(ROWS_PHASE, 128)

    def stack(a, b):
        return jnp.concatenate([r2(a), r2(b)])

    p1s = stack(uu_s, ca_s + N_PAD)
    p1d = stack(uu_d, ca_d)
    p2s = stack(ii_s, cb_s + N_PAD)
    p2d = stack(ii_d, cb_d)
    hp1 = stack(uu_s, ii_d)
    hp2 = stack(uu_d, ca_d)
    hp3 = stack(ii_s, cb_d)

    h1o, h2o, h3o = _hist_call()(hp1, hp2, hp3, zf)
    dq = N_PAD // 128

    def rq(a):
        return a.reshape(dq, 128)

    nuo, nui, nio, nii, ic = _norm_call(
        rq(h1o[:N_PAD]), rq(h2o[:N_PAD]), rq(h3o[:N_PAD]),
        rq(h1o[HIST_PAD:HIST_PAD + N_PAD]),
        rq(h2o[HIST_PAD:HIST_PAD + N_PAD]),
        rq(h3o[HIST_PAD:HIST_PAD + N_PAD]))

    def rcol(a):
        return a.reshape(N_PAD, 1)

    nuo, nui, nio, nii, ic = map(rcol, (nuo, nui, nio, nii, ic))

    hus, his = _pre_call(up, ip, nuo, nio)
    hu, hi = up, ip
    params = [(Wu1, bu1, Wi1, bi1, Ws1, Wn1, bs1),
              (Wu2, bu2, Wi2, bi2, Ws2, Wn2, bs2),
              (Wu3, bu3, Wi3, bi3, Ws3, Wn3, bs3)]
    for l, (Wu, bu, Wi, bi, Ws, Wn, bs) in enumerate(params):
        h1 = jnp.concatenate([hus, hu])
        h2 = jnp.concatenate([his, hu])
        o1, o2 = _seg_call()(h1, h2, p1s, p1d, p2s, p2d, z32)
        su, sca = o1[:N_PAD], o1[N_PAD:]
        si, scb = o2[:N_PAD], o2[N_PAD:]
        final = l == 2
        res = _q_call(final, su, si, sca, scb, hi, nui, nii, ic, nuo, nio,
                      Wu, bu.reshape(1, D), Wi, bi.reshape(1, D),
                      Ws, Wn, bs.reshape(1, D))
        if final:
            return res[0][:N], res[1][:N]
        hu, hi, hus, his = res


# speedup vs baseline: 12.7742x; 1.1305x over previous
import functools

import jax
import jax.numpy as jnp
from jax import lax
from jax.experimental import pallas as pl
from jax.experimental.pallas import tpu as pltpu
from jax.experimental.pallas import tpu_sc as plsc

N = 50000
D = 32
N_PAD = 50048
PAD_IDX = N
E_PHASE = 819200
ROWS_PHASE = 6400
ROWS_SUB = 400
IB = 40
NB = ROWS_SUB // IB
RING = 5
SLAB = N_PAD // 16
HIST_PAD = 65536
SLAB_H = HIST_PAD // 16

@functools.cache
def _mesh():
    return plsc.VectorSubcoreMesh(core_axis_name="c", subcore_axis_name="s",
                                  num_cores=2, num_subcores=16)


def _hist_body(hp1, hp2, hp3, zf, o1, o2, o3, hist, idx_ib, ones_v, sem):
    cid = lax.axis_index("c")
    sid = lax.axis_index("s")
    for k in range(8):
        ones_v[pl.ds(16 * k, 16)] = jnp.full((16,), 1.0, jnp.float32)

    def hphase(hp, out):
        pltpu.sync_copy(zf.at[pl.ds(sid * SLAB_H, SLAB_H)],
                        hist.at[pl.ds(sid * SLAB_H, SLAB_H)])
        plsc.subcore_barrier()
        base = cid * ROWS_PHASE + sid * ROWS_SUB

        @pl.loop(0, NB)
        def _(b):
            pltpu.sync_copy(hp.at[pl.ds(base + b * IB, IB)], idx_ib)

            @pl.loop(0, IB)
            def _(j):
                pltpu.async_copy(ones_v, hist.at[idx_ib.at[j]], sem, add=True)

            @pl.loop(0, IB)
            def _(j):
                pltpu.make_async_copy(ones_v, hist.at[idx_ib.at[0]], sem).wait()
        plsc.subcore_barrier()
        pltpu.sync_copy(hist.at[pl.ds(sid * SLAB_H, SLAB_H)],
                        out.at[pl.ds(cid * HIST_PAD + sid * SLAB_H, SLAB_H)])

    hphase(hp1, o1)
    hphase(hp2, o2)
    hphase(hp3, o3)


_SC_PARAMS = pltpu.CompilerParams(use_tc_tiling_on_sc=False)


@functools.cache
def _hist_call():
    return pl.kernel(
        _hist_body,
        compiler_params=_SC_PARAMS,
        out_type=[jax.ShapeDtypeStruct((2 * HIST_PAD,), jnp.float32)] * 3,
        mesh=_mesh(),
        scratch_types=[
            pltpu.VMEM_SHARED((HIST_PAD,), jnp.float32),
            pltpu.VMEM((IB, 128), jnp.int32),
            pltpu.VMEM((128,), jnp.float32),
            pltpu.SemaphoreType.DMA,
        ],
    )


def _seg_body(h1, h2, p1s, p1d, p2s, p2d, z32, o1, o2,
              acc, sib, dib, rows,
              g0, g1, g2, g3, g4, t0, t1, t2, t3, t4):
    cid = lax.axis_index("c")
    sid = lax.axis_index("s")
    gs = (g0, g1, g2, g3, g4)
    ss = (t0, t1, t2, t3, t4)

    def sphase(h, ps, pd, out):
        pltpu.sync_copy(z32.at[pl.ds(sid * SLAB, SLAB)],
                        acc.at[pl.ds(sid * SLAB, SLAB)])
        plsc.subcore_barrier()
        base = cid * ROWS_PHASE + sid * ROWS_SUB

        @pl.loop(0, NB)
        def _(b):
            row0 = base + b * IB
            pltpu.sync_copy(ps.at[pl.ds(row0, IB)], sib)
            pltpu.sync_copy(pd.at[pl.ds(row0, IB)], dib)
            for c in range(3):
                pltpu.async_copy(h.at[sib.at[c]], rows.at[c], gs[c])
            for jj in (0, 1):
                pltpu.async_copy(h.at[sib.at[jj + 3]], rows.at[jj + 3],
                                 gs[jj + 3])
                pltpu.make_async_copy(h.at[sib.at[0]], rows.at[jj],
                                      gs[jj]).wait()
                pltpu.async_copy(rows.at[jj], acc.at[dib.at[jj]], ss[jj],
                                 add=True)

            @pl.loop(2, IB - 3, step=5)
            def _(j):
                for off in range(5):
                    s = (2 + off) % 5
                    pltpu.make_async_copy(rows.at[off], acc.at[dib.at[0]],
                                          ss[off]).wait()
                    pltpu.async_copy(h.at[sib.at[j + off + 3]], rows.at[off],
                                     gs[off])
                    pltpu.make_async_copy(h.at[sib.at[0]], rows.at[s],
                                          gs[s]).wait()
                    pltpu.async_copy(rows.at[s], acc.at[dib.at[j + off]],
                                     ss[s], add=True)

            for jj in (IB - 3, IB - 2, IB - 1):
                s = jj % 5
                pltpu.make_async_copy(h.at[sib.at[0]], rows.at[s],
                                      gs[s]).wait()
                pltpu.async_copy(rows.at[s], acc.at[dib.at[jj]], ss[s],
                                 add=True)
            for r in range(5):
                pltpu.make_async_copy(rows.at[r], acc.at[dib.at[0]],
                                      ss[r]).wait()
        plsc.subcore_barrier()
        pltpu.sync_copy(acc.at[pl.ds(sid * SLAB, SLAB)],
                        out.at[pl.ds(cid * N_PAD + sid * SLAB, SLAB)])

    sphase(h1, p1s, p1d, o1)
    sphase(h2, p2s, p2d, o2)


@functools.cache
def _seg_call():
    return pl.kernel(
        _seg_body,
        compiler_params=_SC_PARAMS,
        out_type=[jax.ShapeDtypeStruct((2 * N_PAD, D), jnp.float32)] * 2,
        mesh=_mesh(),
        scratch_types=[
            pltpu.VMEM_SHARED((N_PAD, D), jnp.float32),
            pltpu.VMEM((IB, 128), jnp.int32),
            pltpu.VMEM((IB, 128), jnp.int32),
            pltpu.VMEM((RING, 128, D), jnp.float32),
        ] + [pltpu.SemaphoreType.DMA] * 10,
    )


M_PACK = N_PAD // 4
_QROWS = 736
_QGRID = M_PACK // _QROWS


def _norm_kernel_body(duo, dui, dio, dii, ca, cb, nuo, nui, nio, nii, ic):
    for d, o in ((duo, nuo), (dui, nui), (dio, nio), (dii, nii)):
        x = d[...]
        o[...] = jnp.where(x > 0, lax.rsqrt(x), 0.0)
    c = ca[...] + cb[...]
    ic[...] = 1.0 / jnp.maximum(c, 1.0)


def _norm_call(duo, dui, dio, dii, ca, cb):
    blk = pl.BlockSpec((N_PAD // 128, 128), lambda: (0, 0))
    return pl.pallas_call(
        _norm_kernel_body,
        out_shape=[jax.ShapeDtypeStruct((N_PAD // 128, 128), jnp.float32)] * 5,
        in_specs=[blk] * 6,
        out_specs=[blk] * 5,
    )(duo, dui, dio, dii, ca, cb)


def _pre_body(u, i, nuo, nio, hus, his):
    hus[...] = u[...] * nuo[...]
    his[...] = i[...] * nio[...]


def _pre_call(u, i, nuo, nio):
    fb = pl.BlockSpec((_QROWS, 128), lambda r: (r, 0))
    return pl.pallas_call(
        _pre_body,
        grid=(_QGRID,),
        out_shape=[jax.ShapeDtypeStruct((M_PACK, 128), jnp.float32)] * 2,
        in_specs=[fb, fb, fb, fb],
        out_specs=[fb, fb],
    )(u, i, nuo, nio)


def _q_body(final, su, si, ca, cb, hi_in, niu, nii, ic, nuo, nio,
            Wu, bu, Wi, bi, Ws, Wn, bs, *outs):
    f32 = jnp.float32
    ou = jnp.dot(su[...], Wu[...], preferred_element_type=f32) * niu[...] \
        + bu[...]
    mean = (ca[...] + cb[...]) * ic[...]
    oi = jnp.dot(si[...], Wi[...], preferred_element_type=f32) * nii[...] \
        + bi[...] \
        + jnp.dot(hi_in[...], Ws[...], preferred_element_type=f32) \
        + jnp.dot(mean, Wn[...], preferred_element_type=f32) + bs[...]
    if final:
        outs[0][...] = ou
        outs[1][...] = oi
    else:
        hu2 = jnp.maximum(ou, 0.0)
        hi2 = jnp.maximum(oi, 0.0)
        outs[0][...] = hu2
        outs[1][...] = hi2
        outs[2][...] = hu2 * nuo[...]
        outs[3][...] = hi2 * nio[...]


def _q_call(final, su, si, ca, cb, hi_in, niu, nii, ic, nuo, nio,
            Wu, bu, Wi, bi, Ws, Wn, bs):
    fb = pl.BlockSpec((_QROWS, 128), lambda r: (r, 0))
    wb = pl.BlockSpec((128, 128), lambda r: (0, 0))
    bb = pl.BlockSpec((1, 128), lambda r: (0, 0))
    n_out = 2 if final else 4
    return pl.pallas_call(
        functools.partial(_q_body, final),
        grid=(_QGRID,),
        out_shape=[jax.ShapeDtypeStruct((M_PACK, 128), jnp.float32)] * n_out,
        in_specs=[fb, fb, fb, fb, fb, fb, fb, fb, fb, fb,
                  wb, bb, wb, bb, wb, wb, bb],
        out_specs=[fb] * n_out,
    )(su, si, ca, cb, hi_in, niu, nii, ic, nuo, nio,
      Wu, bu, Wi, bi, Ws, Wn, bs)


def _pad_idx(a, e_pad):
    pad = e_pad - a.shape[0]
    return jnp.concatenate([a, jnp.full((pad,), PAD_IDX, jnp.int32)])


def _pad_rows(t):
    return jnp.concatenate([t, jnp.zeros((N_PAD - N, D), t.dtype)])


def kernel(uu_edges, ii_edges, ci_edges, user_table, item_table,
           Wu1, bu1, Wi1, bi1, Ws1, Wn1, bs1,
           Wu2, bu2, Wi2, bi2, Ws2, Wn2, bs2,
           Wu3, bu3, Wi3, bi3, Ws3, Wn3, bs3):
    f32 = jnp.float32
    up = _pad_rows(user_table)
    ip = _pad_rows(item_table)
    z32 = jnp.zeros((N_PAD, D), f32)
    zf = jnp.zeros((HIST_PAD,), f32)

    uu_s = _pad_idx(uu_edges[0], E_PHASE)
    uu_d = _pad_idx(uu_edges[1], E_PHASE)
    ii_s = _pad_idx(ii_edges[0], E_PHASE)
    ii_d = _pad_idx(ii_edges[1], E_PHASE)
    ci_s = _pad_idx(ci_edges[0], 2 * E_PHASE)
    ci_d = _pad_idx(ci_edges[1], 2 * E_PHASE)
    ca_s, cb_s = ci_s[:E_PHASE], ci_s[E_PHASE:]
    ca_d, cb_d = ci_d[:E_PHASE], ci_d[E_PHASE:]

    def r2(a):
        return a.reshape(ROWS_PHASE, 128)

    def stack(a, b):
        return jnp.concatenate([r2(a), r2(b)])

    p1s = stack(uu_s, ca_s + N_PAD)
    p1d = stack(uu_d, ca_d)
    p2s = stack(ii_s, cb_s + N_PAD)
    p2d = stack(ii_d, cb_d)
    hp1 = stack(uu_s, ii_d)
    hp2 = stack(uu_d, ca_d)
    hp3 = stack(ii_s, cb_d)

    h1o, h2o, h3o = _hist_call()(hp1, hp2, hp3, zf)
    dq = N_PAD // 128

    def rq(a):
        return a.reshape(dq, 128)

    nuo, nui, nio, nii, ic = _norm_call(
        rq(h1o[:N_PAD]), rq(h2o[:N_PAD]), rq(h3o[:N_PAD]),
        rq(h1o[HIST_PAD:HIST_PAD + N_PAD]),
        rq(h2o[HIST_PAD:HIST_PAD + N_PAD]),
        rq(h3o[HIST_PAD:HIST_PAD + N_PAD]))

    def packn(a):
        return jnp.repeat(a.reshape(N_PAD), 32).reshape(M_PACK, 128)

    nuo, nui, nio, nii, ic = map(packn, (nuo, nui, nio, nii, ic))

    def packf(t):
        return t.reshape(M_PACK, 128)

    def w4(W):
        return jnp.kron(jnp.eye(4, dtype=f32), W)

    def b4(b):
        return jnp.tile(b, 4).reshape(1, 128)

    hus, his = _pre_call(packf(up), packf(ip), nuo, nio)
    hu, hi = packf(up), packf(ip)
    params = [(Wu1, bu1, Wi1, bi1, Ws1, Wn1, bs1),
              (Wu2, bu2, Wi2, bi2, Ws2, Wn2, bs2),
              (Wu3, bu3, Wi3, bi3, Ws3, Wn3, bs3)]
    for l, (Wu, bu, Wi, bi, Ws, Wn, bs) in enumerate(params):
        h1 = jnp.concatenate([hus, hu]).reshape(2 * N_PAD, D)
        h2 = jnp.concatenate([his, hu]).reshape(2 * N_PAD, D)
        o1, o2 = _seg_call()(h1, h2, p1s, p1d, p2s, p2d, z32)
        o1p = o1.reshape(2 * M_PACK, 128)
        o2p = o2.reshape(2 * M_PACK, 128)
        su, sca = o1p[:M_PACK], o1p[M_PACK:]
        si, scb = o2p[:M_PACK], o2p[M_PACK:]
        final = l == 2
        res = _q_call(final, su, si, sca, scb, hi, nui, nii, ic, nuo, nio,
                      w4(Wu), b4(bu), w4(Wi), b4(bi), w4(Ws), w4(Wn), b4(bs))
        if final:
            return (res[0].reshape(N_PAD, D)[:N],
                    res[1].reshape(N_PAD, D)[:N])
        hu, hi, hus, his = res


# speedup vs baseline: 13.5762x; 1.0628x over previous
import functools

import jax
import jax.numpy as jnp
from jax import lax
from jax.experimental import pallas as pl
from jax.experimental.pallas import tpu as pltpu
from jax.experimental.pallas import tpu_sc as plsc

N = 50000
D = 32
N_PAD = 50048
PAD_IDX = N
E_PHASE = 819200
ROWS_PHASE = 6400
ROWS_SUB = 400
IB = 40
NB = ROWS_SUB // IB
RING = 5
SLAB = N_PAD // 16
HIST_PAD = 65536
SLAB_H = HIST_PAD // 16

@functools.cache
def _mesh():
    return plsc.VectorSubcoreMesh(core_axis_name="c", subcore_axis_name="s",
                                  num_cores=2, num_subcores=16)


def _hist_body(hp1, hp2, hp3, zf, o1, o2, o3, hist, idx_ib, ones_v, sem):
    cid = lax.axis_index("c")
    sid = lax.axis_index("s")
    for k in range(8):
        ones_v[pl.ds(16 * k, 16)] = jnp.full((16,), 1.0, jnp.float32)

    def hphase(hp, out):
        pltpu.sync_copy(zf.at[pl.ds(sid * SLAB_H, SLAB_H)],
                        hist.at[pl.ds(sid * SLAB_H, SLAB_H)])
        plsc.subcore_barrier()
        base = cid * ROWS_PHASE + sid * ROWS_SUB

        @pl.loop(0, NB)
        def _(b):
            pltpu.sync_copy(hp.at[pl.ds(base + b * IB, IB)], idx_ib)

            @pl.loop(0, IB)
            def _(j):
                pltpu.async_copy(ones_v, hist.at[idx_ib.at[j]], sem, add=True)

            @pl.loop(0, IB)
            def _(j):
                pltpu.make_async_copy(ones_v, hist.at[idx_ib.at[0]], sem).wait()
        plsc.subcore_barrier()
        pltpu.sync_copy(hist.at[pl.ds(sid * SLAB_H, SLAB_H)],
                        out.at[pl.ds(cid * HIST_PAD + sid * SLAB_H, SLAB_H)])

    hphase(hp1, o1)
    hphase(hp2, o2)
    hphase(hp3, o3)


_SC_PARAMS = pltpu.CompilerParams(use_tc_tiling_on_sc=False)


@functools.cache
def _hist_call():
    return pl.kernel(
        _hist_body,
        compiler_params=_SC_PARAMS,
        out_type=[jax.ShapeDtypeStruct((2 * HIST_PAD,), jnp.float32)] * 3,
        mesh=_mesh(),
        scratch_types=[
            pltpu.VMEM_SHARED((HIST_PAD,), jnp.float32),
            pltpu.VMEM((IB, 128), jnp.int32),
            pltpu.VMEM((128,), jnp.float32),
            pltpu.SemaphoreType.DMA,
        ],
    )


def _seg_body(h1, h2, p1s, p1d, p2s, p2d, z32, o1, o2,
              acc, sib, dib, rows,
              g0, g1, g2, g3, g4, t0, t1, t2, t3, t4):
    cid = lax.axis_index("c")
    sid = lax.axis_index("s")
    gs = (g0, g1, g2, g3, g4)
    ss = (t0, t1, t2, t3, t4)

    def sphase(h, ps, pd, out):
        pltpu.sync_copy(z32.at[pl.ds(sid * SLAB, SLAB)],
                        acc.at[pl.ds(sid * SLAB, SLAB)])
        plsc.subcore_barrier()
        base = cid * ROWS_PHASE + sid * ROWS_SUB

        @pl.loop(0, NB)
        def _(b):
            row0 = base + b * IB
            pltpu.sync_copy(ps.at[pl.ds(row0, IB)], sib)
            pltpu.sync_copy(pd.at[pl.ds(row0, IB)], dib)
            for c in range(3):
                pltpu.async_copy(h.at[sib.at[c]], rows.at[c], gs[c])
            for jj in (0, 1):
                pltpu.async_copy(h.at[sib.at[jj + 3]], rows.at[jj + 3],
                                 gs[jj + 3])
                pltpu.make_async_copy(h.at[sib.at[0]], rows.at[jj],
                                      gs[jj]).wait()
                pltpu.async_copy(rows.at[jj], acc.at[dib.at[jj]], ss[jj],
                                 add=True)

            @pl.loop(2, IB - 3, step=5)
            def _(j):
                for off in range(5):
                    s = (2 + off) % 5
                    pltpu.make_async_copy(rows.at[off], acc.at[dib.at[0]],
                                          ss[off]).wait()
                    pltpu.async_copy(h.at[sib.at[j + off + 3]], rows.at[off],
                                     gs[off])
                    pltpu.make_async_copy(h.at[sib.at[0]], rows.at[s],
                                          gs[s]).wait()
                    pltpu.async_copy(rows.at[s], acc.at[dib.at[j + off]],
                                     ss[s], add=True)

            for jj in (IB - 3, IB - 2, IB - 1):
                s = jj % 5
                pltpu.make_async_copy(h.at[sib.at[0]], rows.at[s],
                                      gs[s]).wait()
                pltpu.async_copy(rows.at[s], acc.at[dib.at[jj]], ss[s],
                                 add=True)
            for r in range(5):
                pltpu.make_async_copy(rows.at[r], acc.at[dib.at[0]],
                                      ss[r]).wait()
        plsc.subcore_barrier()
        pltpu.sync_copy(acc.at[pl.ds(sid * SLAB, SLAB)],
                        out.at[pl.ds(cid * N_PAD + sid * SLAB, SLAB)])

    sphase(h1, p1s, p1d, o1)
    sphase(h2, p2s, p2d, o2)


def _seg1_body(h, ps, pd, z32, out,
               acc, sib, dib, rows,
               g0, g1, g2, g3, g4, t0, t1, t2, t3, t4):
    cid = lax.axis_index("c")
    sid = lax.axis_index("s")
    gs = (g0, g1, g2, g3, g4)
    ss = (t0, t1, t2, t3, t4)

    pltpu.sync_copy(z32.at[pl.ds(sid * SLAB, SLAB)],
                    acc.at[pl.ds(sid * SLAB, SLAB)])
    plsc.subcore_barrier()
    base = cid * ROWS_PHASE + sid * ROWS_SUB

    @pl.loop(0, NB)
    def _(b):
        row0 = base + b * IB
        pltpu.sync_copy(ps.at[pl.ds(row0, IB)], sib)
        pltpu.sync_copy(pd.at[pl.ds(row0, IB)], dib)
        for c in range(3):
            pltpu.async_copy(h.at[sib.at[c]], rows.at[c], gs[c])
        for jj in (0, 1):
            pltpu.async_copy(h.at[sib.at[jj + 3]], rows.at[jj + 3],
                             gs[jj + 3])
            pltpu.make_async_copy(h.at[sib.at[0]], rows.at[jj],
                                  gs[jj]).wait()
            pltpu.async_copy(rows.at[jj], acc.at[dib.at[jj]], ss[jj],
                             add=True)

        @pl.loop(2, IB - 3, step=5)
        def _(j):
            for off in range(5):
                s = (2 + off) % 5
                pltpu.make_async_copy(rows.at[off], acc.at[dib.at[0]],
                                      ss[off]).wait()
                pltpu.async_copy(h.at[sib.at[j + off + 3]], rows.at[off],
                                 gs[off])
                pltpu.make_async_copy(h.at[sib.at[0]], rows.at[s],
                                      gs[s]).wait()
                pltpu.async_copy(rows.at[s], acc.at[dib.at[j + off]],
                                 ss[s], add=True)

        for jj in (IB - 3, IB - 2, IB - 1):
            s = jj % 5
            pltpu.make_async_copy(h.at[sib.at[0]], rows.at[s],
                                  gs[s]).wait()
            pltpu.async_copy(rows.at[s], acc.at[dib.at[jj]], ss[s],
                             add=True)
        for r in range(5):
            pltpu.make_async_copy(rows.at[r], acc.at[dib.at[0]],
                                  ss[r]).wait()
    plsc.subcore_barrier()
    pltpu.sync_copy(acc.at[pl.ds(sid * SLAB, SLAB)],
                    out.at[pl.ds(cid * N_PAD + sid * SLAB, SLAB)])


@functools.cache
def _seg1_call():
    return pl.kernel(
        _seg1_body,
        compiler_params=_SC_PARAMS,
        out_type=jax.ShapeDtypeStruct((2 * N_PAD, D), jnp.float32),
        mesh=_mesh(),
        scratch_types=[
            pltpu.VMEM_SHARED((N_PAD, D), jnp.float32),
            pltpu.VMEM((IB, 128), jnp.int32),
            pltpu.VMEM((IB, 128), jnp.int32),
            pltpu.VMEM((RING, 128, D), jnp.float32),
        ] + [pltpu.SemaphoreType.DMA] * 10,
    )


@functools.cache
def _seg_call():
    return pl.kernel(
        _seg_body,
        compiler_params=_SC_PARAMS,
        out_type=[jax.ShapeDtypeStruct((2 * N_PAD, D), jnp.float32)] * 2,
        mesh=_mesh(),
        scratch_types=[
            pltpu.VMEM_SHARED((N_PAD, D), jnp.float32),
            pltpu.VMEM((IB, 128), jnp.int32),
            pltpu.VMEM((IB, 128), jnp.int32),
            pltpu.VMEM((RING, 128, D), jnp.float32),
        ] + [pltpu.SemaphoreType.DMA] * 10,
    )


M_PACK = N_PAD // 4
_QROWS = 736
_QGRID = M_PACK // _QROWS


def _norm_kernel_body(duo, dui, dio, dii, ca, cb, nuo, nui, nio, nii, ic):
    for d, o in ((duo, nuo), (dui, nui), (dio, nio), (dii, nii)):
        x = d[...]
        o[...] = jnp.where(x > 0, lax.rsqrt(x), 0.0)
    c = ca[...] + cb[...]
    ic[...] = 1.0 / jnp.maximum(c, 1.0)


def _norm_call(duo, dui, dio, dii, ca, cb):
    blk = pl.BlockSpec((N_PAD // 128, 128), lambda: (0, 0))
    return pl.pallas_call(
        _norm_kernel_body,
        out_shape=[jax.ShapeDtypeStruct((N_PAD // 128, 128), jnp.float32)] * 5,
        in_specs=[blk] * 6,
        out_specs=[blk] * 5,
    )(duo, dui, dio, dii, ca, cb)


def _pre_body(u, i, nuo, nio, hus, his):
    hus[...] = u[...] * nuo[...]
    his[...] = i[...] * nio[...]


def _pre_call(u, i, nuo, nio):
    fb = pl.BlockSpec((_QROWS, 128), lambda r: (r, 0))
    return pl.pallas_call(
        _pre_body,
        grid=(_QGRID,),
        out_shape=[jax.ShapeDtypeStruct((M_PACK, 128), jnp.float32)] * 2,
        in_specs=[fb, fb, fb, fb],
        out_specs=[fb, fb],
    )(u, i, nuo, nio)


def _q_body(final, su, si, ca, cb, hi_in, niu, nii, ic, nuo, nio,
            Wu, bu, Wi, bi, Ws, Wn, bs, *outs):
    f32 = jnp.float32
    ou = jnp.dot(su[...], Wu[...], preferred_element_type=f32) * niu[...] \
        + bu[...]
    mean = (ca[...] + cb[...]) * ic[...]
    oi = jnp.dot(si[...], Wi[...], preferred_element_type=f32) * nii[...] \
        + bi[...] \
        + jnp.dot(hi_in[...], Ws[...], preferred_element_type=f32) \
        + jnp.dot(mean, Wn[...], preferred_element_type=f32) + bs[...]
    if final:
        outs[0][...] = ou
        outs[1][...] = oi
    else:
        hu2 = jnp.maximum(ou, 0.0)
        hi2 = jnp.maximum(oi, 0.0)
        outs[0][...] = hu2
        outs[1][...] = hi2
        outs[2][...] = hu2 * nuo[...]
        outs[3][...] = hi2 * nio[...]


def _q_call(final, su, si, ca, cb, hi_in, niu, nii, ic, nuo, nio,
            Wu, bu, Wi, bi, Ws, Wn, bs):
    fb = pl.BlockSpec((_QROWS, 128), lambda r: (r, 0))
    wb = pl.BlockSpec((128, 128), lambda r: (0, 0))
    bb = pl.BlockSpec((1, 128), lambda r: (0, 0))
    n_out = 2 if final else 4
    out_shape = [jax.ShapeDtypeStruct((M_PACK, 128), jnp.float32)] * n_out
    out_specs = [fb] * n_out
    return pl.pallas_call(
        functools.partial(_q_body, final),
        grid=(_QGRID,),
        out_shape=out_shape,
        in_specs=[fb, fb, fb, fb, fb, fb, fb, fb, fb, fb,
                  wb, bb, wb, bb, wb, wb, bb],
        out_specs=out_specs,
    )(su, si, ca, cb, hi_in, niu, nii, ic, nuo, nio,
      Wu, bu, Wi, bi, Ws, Wn, bs)


def _pad_idx(a, e_pad):
    pad = e_pad - a.shape[0]
    return jnp.concatenate([a, jnp.full((pad,), PAD_IDX, jnp.int32)])


def _pad_rows(t):
    return jnp.concatenate([t, jnp.zeros((N_PAD - N, D), t.dtype)])


def kernel(uu_edges, ii_edges, ci_edges, user_table, item_table,
           Wu1, bu1, Wi1, bi1, Ws1, Wn1, bs1,
           Wu2, bu2, Wi2, bi2, Ws2, Wn2, bs2,
           Wu3, bu3, Wi3, bi3, Ws3, Wn3, bs3):
    f32 = jnp.float32
    up = _pad_rows(user_table)
    ip = _pad_rows(item_table)
    z32 = jnp.zeros((N_PAD, D), f32)
    zf = jnp.zeros((HIST_PAD,), f32)

    uu_s = _pad_idx(uu_edges[0], E_PHASE)
    uu_d = _pad_idx(uu_edges[1], E_PHASE)
    ii_s = _pad_idx(ii_edges[0], E_PHASE)
    ii_d = _pad_idx(ii_edges[1], E_PHASE)
    ci_s = _pad_idx(ci_edges[0], 2 * E_PHASE)
    ci_d = _pad_idx(ci_edges[1], 2 * E_PHASE)
    ca_s, cb_s = ci_s[:E_PHASE], ci_s[E_PHASE:]
    ca_d, cb_d = ci_d[:E_PHASE], ci_d[E_PHASE:]

    def r2(a):
        return a.reshape(ROWS_PHASE, 128)

    def stack(a, b):
        return jnp.concatenate([r2(a), r2(b)])

    p1s = stack(uu_s, ca_s + N_PAD)
    p1d = stack(uu_d, ca_d)
    p2s = stack(ii_s, cb_s + N_PAD)
    p2d = stack(ii_d, cb_d)
    pci_s = stack(ca_s, cb_s)
    pci_d = stack(ca_d, cb_d)
    pui_s = stack(uu_s, ii_s + N_PAD)
    pui_d = stack(uu_d, ii_d)
    hp1 = stack(uu_s, ii_d)
    hp2 = stack(uu_d, ca_d)
    hp3 = stack(ii_s, cb_d)

    h1o, h2o, h3o = _hist_call()(hp1, hp2, hp3, zf)
    dq = N_PAD // 128

    def rq(a):
        return a.reshape(dq, 128)

    nuo, nui, nio, nii, ic = _norm_call(
        rq(h1o[:N_PAD]), rq(h2o[:N_PAD]), rq(h3o[:N_PAD]),
        rq(h1o[HIST_PAD:HIST_PAD + N_PAD]),
        rq(h2o[HIST_PAD:HIST_PAD + N_PAD]),
        rq(h3o[HIST_PAD:HIST_PAD + N_PAD]))

    def packn(a):
        return jnp.repeat(a.reshape(N_PAD), 32).reshape(M_PACK, 128)

    nuo, nui, nio, nii, ic = map(packn, (nuo, nui, nio, nii, ic))

    def packf(t):
        return t.reshape(M_PACK, 128)

    def w4(W):
        return jnp.kron(jnp.eye(4, dtype=f32), W)

    def b4(b):
        return jnp.tile(b, 4).reshape(1, 128)

    hus, his = _pre_call(packf(up), packf(ip), nuo, nio)
    hu, hi = packf(up), packf(ip)
    params = [(Wu1, bu1, Wi1, bi1, Ws1, Wn1, bs1),
              (Wu2, bu2, Wi2, bi2, Ws2, Wn2, bs2),
              (Wu3, bu3, Wi3, bi3, Ws3, Wn3, bs3)]
    for l, (Wu, bu, Wi, bi, Ws, Wn, bs) in enumerate(params):
        if l == 0:
            o_c = _seg1_call()(up, pci_s, pci_d, z32)
            h12 = jnp.concatenate([hus, his]).reshape(2 * N_PAD, D)
            o_ui = _seg1_call()(h12, pui_s, pui_d, z32)
            ocp = o_c.reshape(2 * M_PACK, 128)
            ouip = o_ui.reshape(2 * M_PACK, 128)
            su, si = ouip[:M_PACK], ouip[M_PACK:]
            sca, scb = ocp[:M_PACK], ocp[M_PACK:]
        else:
            h1 = jnp.concatenate([hus, hu]).reshape(2 * N_PAD, D)
            h2 = jnp.concatenate([his, hu]).reshape(2 * N_PAD, D)
            o1, o2 = _seg_call()(h1, h2, p1s, p1d, p2s, p2d, z32)
            o1p = o1.reshape(2 * M_PACK, 128)
            o2p = o2.reshape(2 * M_PACK, 128)
            su, sca = o1p[:M_PACK], o1p[M_PACK:]
            si, scb = o2p[:M_PACK], o2p[M_PACK:]
        final = l == 2
        res = _q_call(final, su, si, sca, scb, hi, nui, nii, ic, nuo, nio,
                      w4(Wu), b4(bu), w4(Wi), b4(bi), w4(Ws), w4(Wn), b4(bs))
        if final:
            return (res[0].reshape(N_PAD, D)[:N],
                    res[1].reshape(N_PAD, D)[:N])
        hu, hi, hus, his = res
